# Initial kernel scaffold; baseline (speedup 1.0000x reference)
import jax
import jax.numpy as jnp
from jax.experimental import pallas as pl


def kernel(x, router_logits, w13_weight, w2_weight):
    raise NotImplementedError("write your pallas kernel here")


# speedup vs baseline: 1.4734x; 1.4734x over previous
import functools
import jax
import jax.numpy as jnp
from jax import lax
from jax.experimental import pallas as pl
from jax.experimental.pallas import tpu as pltpu

E = 8
TOPK = 2
H = 1024
I = 2048
T = 2048
B = 256
NPAD = 5888
NB = NPAD // B


def _routing(router_logits):
    probs = jax.nn.softmax(router_logits.astype(jnp.float32), axis=-1)
    topw, topi = jax.lax.top_k(probs, TOPK)
    topw = topw / jnp.sum(topw, axis=-1, keepdims=True)
    return topw, topi.astype(jnp.int32)


def _dispatch(topi):
    N = T * TOPK
    flat_e = topi.reshape(-1)
    perm = jnp.argsort(flat_e, stable=True)
    sorted_e = flat_e[perm]
    counts = jnp.bincount(flat_e, length=E)
    offs = jnp.concatenate([jnp.zeros((1,), jnp.int32),
                            jnp.cumsum(counts)[:-1].astype(jnp.int32)])
    padded = ((counts + B - 1) // B) * B
    bounds = jnp.cumsum(padded).astype(jnp.int32)
    poffs = jnp.concatenate([jnp.zeros((1,), jnp.int32), bounds[:-1]])
    dest = (jnp.arange(N, dtype=jnp.int32) - offs[sorted_e] + poffs[sorted_e])
    order = jnp.zeros((NPAD,), jnp.int32).at[dest].set(
        (perm // TOPK).astype(jnp.int32))
    pos = jnp.zeros((N,), jnp.int32).at[perm].set(dest).reshape(T, TOPK)
    bstart = jnp.arange(32, dtype=jnp.int32) * B
    eb = jnp.minimum((bstart[:, None] >= bounds[None, :]).sum(-1), E - 1)
    num_active = (bounds[-1] // B).astype(jnp.int32)
    block_expert = eb.astype(jnp.int32).at[31].set(num_active)
    return order, pos, block_expert


def _gather_rows(x, order):
    order = jnp.clip(order, 0, T - 1)
    return x[order]


def _ffn_body(be_ref, xs_ref, w13_ref, w2_ref, y_ref):
    b = pl.program_id(0)

    @pl.when(b < be_ref[31])
    def _():
        xb = xs_ref[...]
        h = jax.lax.dot_general(xb, w13_ref[0], (((1,), (1,)), ((), ())),
                                preferred_element_type=jnp.float32)
        g = h[:, :I]
        u = h[:, I:]
        act = g * jax.nn.sigmoid(g) * u
        y_ref[...] = jax.lax.dot_general(act, w2_ref[0],
                                         (((1,), (1,)), ((), ())),
                                         preferred_element_type=jnp.float32)


def _ffn_tc(block_expert, xs, w13_weight, w2_weight, interpret=False):
    grid_spec = pltpu.PrefetchScalarGridSpec(
        num_scalar_prefetch=1,
        grid=(NB,),
        in_specs=[
            pl.BlockSpec((B, H), lambda b, be: (b, 0)),
            pl.BlockSpec((1, 2 * I, H), lambda b, be: (be[b], 0, 0)),
            pl.BlockSpec((1, H, I), lambda b, be: (be[b], 0, 0)),
        ],
        out_specs=pl.BlockSpec((B, H), lambda b, be: (b, 0)),
    )
    return pl.pallas_call(
        _ffn_body,
        grid_spec=grid_spec,
        out_shape=jax.ShapeDtypeStruct((NPAD, H), jnp.float32),
        compiler_params=pltpu.CompilerParams(
            dimension_semantics=("arbitrary",),
            vmem_limit_bytes=128 * 1024 * 1024,
        ),
        interpret=interpret,
    )(block_expert, xs, w13_weight, w2_weight)


def _combine(y, pos, gates):
    r0 = y[pos[:, 0]]
    r1 = y[pos[:, 1]]
    return gates[:, 0:1] * r0 + gates[:, 1:2] * r1


def kernel(x, router_logits, w13_weight, w2_weight):
    gates, topi = _routing(router_logits)
    order, pos, block_expert = _dispatch(topi)
    xs = _gather_rows(x, order)
    y = _ffn_tc(block_expert, xs, w13_weight, w2_weight)
    return _combine(y, pos, gates)
